# baseline (speedup 1.0000x reference)
import functools

import jax
import jax.numpy as jnp
from jax import lax
from jax.experimental import pallas as pl
from jax.experimental.pallas import tpu as pltpu
from jax.experimental.pallas import tpu_sc as plsc

N = 50000
E = 1600000
G = 1024
IN = 38
INP = 64
H = 128
OUT = 67
EPS = 1e-5

BLK = 2000
NBLK = N // BLK

DEGP = 50048
DEGB = 2 * DEGP


def _vsmesh():
    return plsc.VectorSubcoreMesh(core_axis_name="c", subcore_axis_name="s")


_DEG_EPC = E // 2
_DEG_EPT = _DEG_EPC // 16
_DEG_CH = 4096
_DEG_NCH = _DEG_EPT // _DEG_CH
_DEG_TAIL = _DEG_EPT - _DEG_NCH * _DEG_CH
_DEG_TAILP = 896
_DEG_SL = DEGB // 16


@functools.partial(
    pl.kernel,
    out_type=jax.ShapeDtypeStruct((2 * DEGB,), jnp.float32),
    mesh=_vsmesh(),
    scratch_types=[
        pltpu.VMEM((_DEG_CH,), jnp.int32),
        pltpu.VMEM((128,), jnp.int32),
        pltpu.VMEM((128,), jnp.float32),
        pltpu.VMEM((_DEG_SL,), jnp.float32),
        pltpu.VMEM_SHARED((DEGB,), jnp.float32),
    ],
)
def _deg_kernel(src_h, dst_h, out_h, idxb, idxg, onesb, stage, acc):
    c = lax.axis_index("c")
    s = lax.axis_index("s")
    iota16 = lax.iota(jnp.int32, 16)
    ones16 = jnp.ones((16,), jnp.float32)
    zeros16 = jnp.zeros((16,), jnp.float32)
    for j in range(8):
        onesb[pl.ds(j * 16, 16)] = ones16

    def zbody(j, _):
        stage[pl.ds(j * 16, 16)] = zeros16
        return 0
    lax.fori_loop(0, _DEG_SL // 16, zbody, 0)
    pltpu.sync_copy(stage, acc.at[pl.ds(s * _DEG_SL, _DEG_SL)])
    plsc.subcore_barrier()
    ebase = c * _DEG_EPC + s * _DEG_EPT

    def do_groups(n_groups, off):
        def gbody(g, _):
            def cb(j, _2):
                v = idxb[pl.ds(g * 128 + j * 16, 16)]
                idxg[pl.ds(j * 16, 16)] = v + off
                return 0
            lax.fori_loop(0, 8, cb, 0)
            pltpu.sync_copy(onesb, acc.at[idxg], add=True)
            return 0
        lax.fori_loop(0, n_groups, gbody, 0)

    def chbody(ch, _):
        cb0 = ebase + ch * _DEG_CH
        pltpu.sync_copy(src_h.at[pl.ds(cb0, _DEG_CH)], idxb)
        do_groups(_DEG_CH // 128, 0)
        pltpu.sync_copy(dst_h.at[pl.ds(cb0, _DEG_CH)], idxb)
        do_groups(_DEG_CH // 128, DEGP)
        return 0
    lax.fori_loop(0, _DEG_NCH, chbody, 0)

    tb = ebase + _DEG_NCH * _DEG_CH
    for arr_h, off in ((src_h, 0), (dst_h, DEGP)):
        pltpu.sync_copy(arr_h.at[pl.ds(tb, _DEG_TAIL)],
                        idxb.at[pl.ds(0, _DEG_TAIL)])
        for j in range((_DEG_TAILP - _DEG_TAIL) // 16):
            idxb[pl.ds(_DEG_TAIL + j * 16, 16)] = N + iota16
        do_groups(_DEG_TAILP // 128, off)

    plsc.subcore_barrier()
    pltpu.sync_copy(acc.at[pl.ds(s * _DEG_SL, _DEG_SL)], stage)
    pltpu.sync_copy(stage, out_h.at[pl.ds(c * DEGB + s * _DEG_SL, _DEG_SL)])


EROWS = 12800
_SC_RPT = EROWS // 16
_SC_CHG = 32
_SC_NCH = _SC_RPT // _SC_CHG
NP2 = 50048
_SC_WRT = NP2 // 16
COLW = 32


def _make_edge_scatter(NBLKT):
    npass = NBLKT // 2

    @functools.partial(
        pl.kernel,
        out_type=[jax.ShapeDtypeStruct((NP2, COLW), jnp.float32)
                  for _ in range(NBLKT)],
        mesh=_vsmesh(),
        scratch_types=(
            [pltpu.VMEM_SHARED((NP2, COLW), jnp.float32)]
            + [pltpu.VMEM((_SC_CHG * 128,), jnp.int32) for _ in range(2)]
            + [pltpu.VMEM((_SC_CHG * 128,), jnp.int32) for _ in range(2)]
            + [pltpu.VMEM((128, COLW), jnp.float32) for _ in range(2)]
            + [pltpu.SemaphoreType.DMA for _ in range(3)]
        ),
        compiler_params=pltpu.CompilerParams(
            use_tc_tiling_on_sc=False, internal_scratch_in_bytes=131072),
    )
    def edge_scatter(src_h, dst_h, *rest):
        tabs = rest[:NBLKT]
        outs = rest[NBLKT:2 * NBLKT]
        sc = rest[2 * NBLKT:]
        acc = sc[0]
        srcb = sc[1:3]
        dstb = sc[3:5]
        rows = sc[5:7]
        csem = sc[7]
        gsem = sc[8:10]
        c = lax.axis_index("c")
        s = lax.axis_index("s")
        zeros16 = jnp.zeros((16,), jnp.float32)
        rbase = s * _SC_RPT

        CE = _SC_CHG * 128

        def start_chunk_dma(ch, b):
            e0 = (rbase + ch * _SC_CHG) * 128
            pltpu.async_copy(src_h.at[pl.ds(e0, CE)], srcb[b], csem)
            pltpu.async_copy(dst_h.at[pl.ds(e0, CE)], dstb[b], csem)

        def wait_chunk_dma(b):
            pltpu.make_async_copy(src_h.at[pl.ds(0, CE)], srcb[b],
                                  csem).wait()
            pltpu.make_async_copy(dst_h.at[pl.ds(0, CE)], dstb[b],
                                  csem).wait()

        def chunk(tab, ch, b):
            wait_chunk_dma(b)

            @pl.when(ch + 1 < _SC_NCH)
            def _():
                start_chunk_dma(ch + 1, 1 - b)

            sb, db = srcb[b], dstb[b]

            def sidx(g):
                return sb.at[pl.ds(g * 128, 128)]

            def didx(g):
                return db.at[pl.ds(g * 128, 128)]

            pltpu.async_copy(tab.at[sidx(0)], rows[0], gsem[0])

            def gbody(k, _):
                for j in range(2):
                    g = 2 * k + j
                    bf = j
                    pltpu.make_async_copy(tab.at[sidx(g)], rows[bf],
                                          gsem[bf]).wait()

                    @pl.when(g + 1 < _SC_CHG)
                    def _():
                        pltpu.async_copy(tab.at[sidx(g + 1)], rows[1 - bf],
                                         gsem[1 - bf])
                    pltpu.sync_copy(rows[bf], acc.at[didx(g)], add=True)
                return 0
            lax.fori_loop(0, _SC_CHG // 2, gbody, 0)

        def edges(tab):
            start_chunk_dma(0, 0)

            def ch2(m, _):
                chunk(tab, 2 * m, 0)
                chunk(tab, 2 * m + 1, 1)
                return 0
            lax.fori_loop(0, _SC_NCH // 2, ch2, 0)
            chunk(tab, _SC_NCH - 1, 0)

        def writeout(out_h):
            woff = 0
            while woff < _SC_WRT:
                sz = min(128, _SC_WRT - woff)
                pltpu.sync_copy(acc.at[pl.ds(s * _SC_WRT + woff, sz)],
                                rows[0].at[pl.ds(0, sz)])
                pltpu.sync_copy(rows[0].at[pl.ds(0, sz)],
                                out_h.at[pl.ds(s * _SC_WRT + woff, sz)])
                woff += sz

        for p in range(npass):
            def zbody(i, _):
                for j in range(COLW // 16):
                    rows[0][i, pl.ds(j * 16, 16)] = zeros16
                return 0
            lax.fori_loop(0, 128, zbody, 0)
            zoff = 0
            while zoff < _SC_WRT:
                sz = min(128, _SC_WRT - zoff)
                pltpu.sync_copy(rows[0].at[pl.ds(0, sz)],
                                acc.at[pl.ds(s * _SC_WRT + zoff, sz)])
                zoff += sz
            plsc.subcore_barrier()

            @pl.when(c == 0)
            def _():
                edges(tabs[2 * p])

            @pl.when(c == 1)
            def _():
                edges(tabs[2 * p + 1])
            plsc.subcore_barrier()

            @pl.when(c == 0)
            def _():
                writeout(outs[2 * p])

            @pl.when(c == 1)
            def _():
                writeout(outs[2 * p + 1])
            plsc.subcore_barrier()

    return edge_scatter


_edge_scatter2 = _make_edge_scatter(2)
_edge_scatter4 = _make_edge_scatter(4)


def _k0_body(f_ref, po0, po1, pi0, pi1, t0_ref, t1_ref, ns_ref, nd_ref):
    dout = po0[...] + po1[...]
    din = pi0[...] + pi1[...]
    ns = jnp.where(dout > 0, lax.rsqrt(jnp.maximum(dout, 1.0)), 0.0)
    nd = jnp.where(din > 0, lax.rsqrt(jnp.maximum(din, 1.0)), 0.0)
    ns_ref[...] = ns
    nd_ref[...] = nd
    t = f_ref[...] * ns
    for q, r in enumerate((t0_ref, t1_ref)):
        r[...] = t[:, q * COLW:(q + 1) * COLW]


_k0 = pl.pallas_call(
    _k0_body,
    grid=(NBLK,),
    in_specs=[
        pl.BlockSpec((BLK, INP), lambda i: (i, 0)),
        pl.BlockSpec((BLK, 1), lambda i: (i, 0)),
        pl.BlockSpec((BLK, 1), lambda i: (i, 0)),
        pl.BlockSpec((BLK, 1), lambda i: (i, 0)),
        pl.BlockSpec((BLK, 1), lambda i: (i, 0)),
    ],
    out_specs=[pl.BlockSpec((BLK, COLW), lambda i: (i, 0))] * 2
    + [
        pl.BlockSpec((BLK, 1), lambda i: (i, 0)),
        pl.BlockSpec((BLK, 1), lambda i: (i, 0)),
    ],
    out_shape=[jax.ShapeDtypeStruct((N, COLW), jnp.float32)] * 2
    + [
        jax.ShapeDtypeStruct((N, 1), jnp.float32),
        jax.ShapeDtypeStruct((N, 1), jnp.float32),
    ],
)


def _make_ka(nparts):
    Wd = nparts * COLW

    def body(*refs):
        us = refs[:nparts]
        nd_ref, w_ref, b_ref, z_ref, st_ref = refs[nparts:]
        u = jnp.concatenate([r[...] for r in us], axis=1)
        z = jnp.dot(u * nd_ref[...], w_ref[...],
                    preferred_element_type=jnp.float32,
                    precision=lax.Precision.HIGHEST) + b_ref[...]
        z_ref[...] = z

        @pl.when(pl.program_id(0) == 0)
        def _():
            st_ref[...] = jnp.zeros((1, H), jnp.float32)

        st_ref[...] += jnp.sum(z, 0, keepdims=True)

    return pl.pallas_call(
        body,
        grid=(NBLK,),
        in_specs=[pl.BlockSpec((BLK, COLW), lambda i: (i, 0))] * nparts
        + [
            pl.BlockSpec((BLK, 1), lambda i: (i, 0)),
            pl.BlockSpec((Wd, H), lambda i: (0, 0)),
            pl.BlockSpec((1, H), lambda i: (0, 0)),
        ],
        out_specs=[
            pl.BlockSpec((BLK, H), lambda i: (i, 0)),
            pl.BlockSpec((1, H), lambda i: (0, 0)),
        ],
        out_shape=[
            jax.ShapeDtypeStruct((N, H), jnp.float32),
            jax.ShapeDtypeStruct((1, H), jnp.float32),
        ],
    )


_ka64 = _make_ka(2)
_ka128 = _make_ka(4)


def _kv_body(z_ref, st_ref, vs_ref):
    mu = st_ref[...] * (1.0 / N)
    d = z_ref[...] - mu

    @pl.when(pl.program_id(0) == 0)
    def _():
        vs_ref[...] = jnp.zeros((1, H), jnp.float32)

    vs_ref[...] += jnp.sum(d * d, 0, keepdims=True)


_kv = pl.pallas_call(
    _kv_body,
    grid=(NBLK,),
    in_specs=[
        pl.BlockSpec((BLK, H), lambda i: (i, 0)),
        pl.BlockSpec((1, H), lambda i: (0, 0)),
    ],
    out_specs=pl.BlockSpec((1, H), lambda i: (0, 0)),
    out_shape=jax.ShapeDtypeStruct((1, H), jnp.float32),
)


def _bn_coeffs(st, vs, g, bb):
    mu = st * (1.0 / N)
    var = vs * (1.0 / N)
    a = g * lax.rsqrt(var + EPS)
    cc = bb - mu * a
    return a, cc


def _kb_mid_body(z_ref, st_ref, vs_ref, ns_ref, g_ref, bb_ref,
                 t0_ref, t1_ref, t2_ref, t3_ref):
    a, cc = _bn_coeffs(st_ref[...], vs_ref[...], g_ref[...], bb_ref[...])
    y = jnp.maximum(z_ref[...] * a + cc, 0.0)
    t = y * ns_ref[...]
    cw = H // 4
    for q, r in enumerate((t0_ref, t1_ref, t2_ref, t3_ref)):
        r[...] = t[:, q * cw:(q + 1) * cw]


_kb_mid = pl.pallas_call(
    _kb_mid_body,
    grid=(NBLK,),
    in_specs=[
        pl.BlockSpec((BLK, H), lambda i: (i, 0)),
        pl.BlockSpec((1, H), lambda i: (0, 0)),
        pl.BlockSpec((1, H), lambda i: (0, 0)),
        pl.BlockSpec((BLK, 1), lambda i: (i, 0)),
        pl.BlockSpec((1, H), lambda i: (0, 0)),
        pl.BlockSpec((1, H), lambda i: (0, 0)),
    ],
    out_specs=[pl.BlockSpec((BLK, H // 4), lambda i: (i, 0))] * 4,
    out_shape=[jax.ShapeDtypeStruct((N, H // 4), jnp.float32)] * 4,
)


def _kb_fin_body(z_ref, st_ref, vs_ref, g_ref, bb_ref, aww_ref, awb_ref,
                 gid_ref, aw_ref, seg_ref):
    a, cc = _bn_coeffs(st_ref[...], vs_ref[...], g_ref[...], bb_ref[...])
    y = jnp.maximum(z_ref[...] * a + cc, 0.0)
    aw = jnp.sum(y * aww_ref[...], axis=1, keepdims=True) + awb_ref[...]
    aw_ref[...] = aw
    w = 1.0 / (1.0 + jnp.exp(-aw))
    hw = y * w
    oh = (gid_ref[...] == lax.broadcasted_iota(jnp.int32, (BLK, G), 1)
          ).astype(jnp.float32)

    @pl.when(pl.program_id(0) == 0)
    def _():
        seg_ref[...] = jnp.zeros((G, H), jnp.float32)

    seg_ref[...] += lax.dot_general(oh, hw, (((0,), (0,)), ((), ())),
                                    preferred_element_type=jnp.float32,
                                    precision=lax.Precision.HIGHEST)


_kb_fin = pl.pallas_call(
    _kb_fin_body,
    grid=(NBLK,),
    in_specs=[
        pl.BlockSpec((BLK, H), lambda i: (i, 0)),
        pl.BlockSpec((1, H), lambda i: (0, 0)),
        pl.BlockSpec((1, H), lambda i: (0, 0)),
        pl.BlockSpec((1, H), lambda i: (0, 0)),
        pl.BlockSpec((1, H), lambda i: (0, 0)),
        pl.BlockSpec((1, H), lambda i: (0, 0)),
        pl.BlockSpec((1, 1), lambda i: (0, 0)),
        pl.BlockSpec((BLK, 1), lambda i: (i, 0)),
    ],
    out_specs=[
        pl.BlockSpec((BLK, 1), lambda i: (i, 0)),
        pl.BlockSpec((G, H), lambda i: (0, 0)),
    ],
    out_shape=[
        jax.ShapeDtypeStruct((N, 1), jnp.float32),
        jax.ShapeDtypeStruct((G, H), jnp.float32),
    ],
)


def _head_body(seg_ref, w1_ref, b1_ref, g1_ref, c1_ref, w2_ref, b2_ref,
               g2_ref, c2_ref, w3_ref, b3_ref, o_ref):
    x = jnp.dot(seg_ref[...], w1_ref[...], preferred_element_type=jnp.float32,
                precision=lax.Precision.HIGHEST) + b1_ref[...]
    mu = jnp.mean(x, 0, keepdims=True)
    d = x - mu
    var = jnp.mean(d * d, 0, keepdims=True)
    x = jnp.maximum(d * lax.rsqrt(var + EPS) * g1_ref[...] + c1_ref[...], 0.0)
    x = jnp.dot(x, w2_ref[...], preferred_element_type=jnp.float32,
                precision=lax.Precision.HIGHEST) + b2_ref[...]
    mu = jnp.mean(x, 0, keepdims=True)
    d = x - mu
    var = jnp.mean(d * d, 0, keepdims=True)
    x = jnp.maximum(d * lax.rsqrt(var + EPS) * g2_ref[...] + c2_ref[...], 0.0)
    x = jnp.dot(x, w3_ref[...], preferred_element_type=jnp.float32,
                precision=lax.Precision.HIGHEST) + b3_ref[...]
    o_ref[...] = 1.0 / (1.0 + jnp.exp(-x))


_head = pl.pallas_call(
    _head_body,
    out_shape=jax.ShapeDtypeStruct((G, H), jnp.float32),
)


def kernel(feats, edge_index, node_graph_ids,
           gcn1_W, gcn1_b, gcn1_bn_g, gcn1_bn_b,
           gcn2_0_W, gcn2_0_b, gcn2_0_bn_g, gcn2_0_bn_b,
           gcn2_1_W, gcn2_1_b, gcn2_1_bn_g, gcn2_1_bn_b,
           aw_W, aw_b, fc1_W, fc1_b, bn1_g, bn1_b,
           lin0_W, lin0_b, bnl0_g, bnl0_b, fc2_W, fc2_b):
    src = edge_index[0]
    dst = edge_index[1]
    feats64 = jnp.pad(feats, ((0, 0), (0, INP - IN)))
    W1p = jnp.pad(gcn1_W, ((0, INP - IN), (0, 0)))
    gids2 = node_graph_ids.reshape(N, 1)

    npad = EROWS * 128 - E
    pidx = jnp.arange(npad, dtype=jnp.int32)
    src2 = jnp.concatenate([src, pidx % 128])
    dst2 = jnp.concatenate([dst, N + pidx % 48])

    degflat = _deg_kernel(src, dst)
    degr = degflat.reshape(2, 2, DEGP)
    po0 = degr[0, 0, :N].reshape(N, 1)
    pi0 = degr[0, 1, :N].reshape(N, 1)
    po1 = degr[1, 0, :N].reshape(N, 1)
    pi1 = degr[1, 1, :N].reshape(N, 1)

    t0a, t0b, ns, nd = _k0(feats64, po0, po1, pi0, pi1)

    u1 = _edge_scatter2(src2, dst2, t0a, t0b)
    z1, st1 = _ka64(*u1, nd, W1p, gcn1_b.reshape(1, H))
    vs1 = _kv(z1, st1)
    t1 = _kb_mid(z1, st1, vs1, ns, gcn1_bn_g.reshape(1, H),
                 gcn1_bn_b.reshape(1, H))

    u2 = _edge_scatter4(src2, dst2, *t1)
    z2, st2 = _ka128(*u2, nd, gcn2_0_W, gcn2_0_b.reshape(1, H))
    vs2 = _kv(z2, st2)
    t2 = _kb_mid(z2, st2, vs2, ns, gcn2_0_bn_g.reshape(1, H),
                 gcn2_0_bn_b.reshape(1, H))

    u3 = _edge_scatter4(src2, dst2, *t2)
    z3, st3 = _ka128(*u3, nd, gcn2_1_W, gcn2_1_b.reshape(1, H))
    vs3 = _kv(z3, st3)
    aw, seg = _kb_fin(z3, st3, vs3, gcn2_1_bn_g.reshape(1, H),
                      gcn2_1_bn_b.reshape(1, H), aw_W.reshape(1, H),
                      aw_b.reshape(1, 1), gids2)

    w3p = jnp.pad(fc2_W, ((0, 0), (0, H - OUT)))
    b3p = jnp.pad(fc2_b, ((0, H - OUT))).reshape(1, H)
    headp = _head(seg, fc1_W, fc1_b.reshape(1, 256), bn1_g.reshape(1, 256),
                  bn1_b.reshape(1, 256), lin0_W, lin0_b.reshape(1, H),
                  bnl0_g.reshape(1, H), bnl0_b.reshape(1, H), w3p, b3p)
    x = headp[:, :OUT]
    return (x, aw)


# speedup vs baseline: 1.0006x; 1.0006x over previous
import functools

import jax
import jax.numpy as jnp
from jax import lax
from jax.experimental import pallas as pl
from jax.experimental.pallas import tpu as pltpu
from jax.experimental.pallas import tpu_sc as plsc

N = 50000
E = 1600000
G = 1024
IN = 38
INP = 64
H = 128
OUT = 67
EPS = 1e-5

BLK = 2000
NBLK = N // BLK

DEGP = 50048
DEGB = 2 * DEGP


def _vsmesh():
    return plsc.VectorSubcoreMesh(core_axis_name="c", subcore_axis_name="s")


_DEG_EPC = E // 2
_DEG_EPT = _DEG_EPC // 16
_DEG_CH = 4096
_DEG_NCH = _DEG_EPT // _DEG_CH
_DEG_TAIL = _DEG_EPT - _DEG_NCH * _DEG_CH
_DEG_TAILP = 896
_DEG_SL = DEGB // 16


@functools.partial(
    pl.kernel,
    out_type=jax.ShapeDtypeStruct((2 * DEGB,), jnp.float32),
    mesh=_vsmesh(),
    scratch_types=[
        pltpu.VMEM((_DEG_CH,), jnp.int32),
        pltpu.VMEM((128,), jnp.int32),
        pltpu.VMEM((128,), jnp.float32),
        pltpu.VMEM((_DEG_SL,), jnp.float32),
        pltpu.VMEM_SHARED((DEGB,), jnp.float32),
    ],
)
def _deg_kernel(src_h, dst_h, out_h, idxb, idxg, onesb, stage, acc):
    c = lax.axis_index("c")
    s = lax.axis_index("s")
    iota16 = lax.iota(jnp.int32, 16)
    ones16 = jnp.ones((16,), jnp.float32)
    zeros16 = jnp.zeros((16,), jnp.float32)
    for j in range(8):
        onesb[pl.ds(j * 16, 16)] = ones16

    def zbody(j, _):
        stage[pl.ds(j * 16, 16)] = zeros16
        return 0
    lax.fori_loop(0, _DEG_SL // 16, zbody, 0)
    pltpu.sync_copy(stage, acc.at[pl.ds(s * _DEG_SL, _DEG_SL)])
    plsc.subcore_barrier()
    ebase = c * _DEG_EPC + s * _DEG_EPT

    def do_groups(n_groups, off):
        def gbody(g, _):
            def cb(j, _2):
                v = idxb[pl.ds(g * 128 + j * 16, 16)]
                idxg[pl.ds(j * 16, 16)] = v + off
                return 0
            lax.fori_loop(0, 8, cb, 0)
            pltpu.sync_copy(onesb, acc.at[idxg], add=True)
            return 0
        lax.fori_loop(0, n_groups, gbody, 0)

    def chbody(ch, _):
        cb0 = ebase + ch * _DEG_CH
        pltpu.sync_copy(src_h.at[pl.ds(cb0, _DEG_CH)], idxb)
        do_groups(_DEG_CH // 128, 0)
        pltpu.sync_copy(dst_h.at[pl.ds(cb0, _DEG_CH)], idxb)
        do_groups(_DEG_CH // 128, DEGP)
        return 0
    lax.fori_loop(0, _DEG_NCH, chbody, 0)

    tb = ebase + _DEG_NCH * _DEG_CH
    for arr_h, off in ((src_h, 0), (dst_h, DEGP)):
        pltpu.sync_copy(arr_h.at[pl.ds(tb, _DEG_TAIL)],
                        idxb.at[pl.ds(0, _DEG_TAIL)])
        for j in range((_DEG_TAILP - _DEG_TAIL) // 16):
            idxb[pl.ds(_DEG_TAIL + j * 16, 16)] = N + iota16
        do_groups(_DEG_TAILP // 128, off)

    plsc.subcore_barrier()
    pltpu.sync_copy(acc.at[pl.ds(s * _DEG_SL, _DEG_SL)], stage)
    pltpu.sync_copy(stage, out_h.at[pl.ds(c * DEGB + s * _DEG_SL, _DEG_SL)])


EROWS = 12800
_SC_RPT = EROWS // 16
_SC_CHG = 32
_SC_NCH = _SC_RPT // _SC_CHG
NP2 = 50048
_SC_WRT = NP2 // 16
COLW = 32


def _make_edge_scatter(NBLKT):
    npass = NBLKT // 2

    @functools.partial(
        pl.kernel,
        out_type=[jax.ShapeDtypeStruct((NP2, COLW), jnp.float32)
                  for _ in range(NBLKT)],
        mesh=_vsmesh(),
        scratch_types=(
            [pltpu.VMEM_SHARED((NP2, COLW), jnp.float32)]
            + [pltpu.VMEM((_SC_CHG * 128,), jnp.int32) for _ in range(2)]
            + [pltpu.VMEM((_SC_CHG * 128,), jnp.int32) for _ in range(2)]
            + [pltpu.VMEM((128, COLW), jnp.float32) for _ in range(2)]
            + [pltpu.SemaphoreType.DMA for _ in range(3)]
        ),
        compiler_params=pltpu.CompilerParams(use_tc_tiling_on_sc=False),
    )
    def edge_scatter(src_h, dst_h, *rest):
        tabs = rest[:NBLKT]
        outs = rest[NBLKT:2 * NBLKT]
        sc = rest[2 * NBLKT:]
        acc = sc[0]
        srcb = sc[1:3]
        dstb = sc[3:5]
        rows = sc[5:7]
        csem = sc[7]
        gsem = sc[8:10]
        c = lax.axis_index("c")
        s = lax.axis_index("s")
        zeros16 = jnp.zeros((16,), jnp.float32)
        rbase = s * _SC_RPT

        CE = _SC_CHG * 128

        def start_chunk_dma(ch, b):
            e0 = (rbase + ch * _SC_CHG) * 128
            pltpu.async_copy(src_h.at[pl.ds(e0, CE)], srcb[b], csem)
            pltpu.async_copy(dst_h.at[pl.ds(e0, CE)], dstb[b], csem)

        def wait_chunk_dma(b):
            pltpu.make_async_copy(src_h.at[pl.ds(0, CE)], srcb[b],
                                  csem).wait()
            pltpu.make_async_copy(dst_h.at[pl.ds(0, CE)], dstb[b],
                                  csem).wait()

        def chunk(tab, ch, b):
            wait_chunk_dma(b)

            @pl.when(ch + 1 < _SC_NCH)
            def _():
                start_chunk_dma(ch + 1, 1 - b)

            sb, db = srcb[b], dstb[b]

            def sidx(g):
                return sb.at[pl.ds(g * 128, 128)]

            def didx(g):
                return db.at[pl.ds(g * 128, 128)]

            pltpu.async_copy(tab.at[sidx(0)], rows[0], gsem[0])

            def gbody(k, _):
                for j in range(2):
                    g = 2 * k + j
                    bf = j
                    pltpu.make_async_copy(tab.at[sidx(g)], rows[bf],
                                          gsem[bf]).wait()

                    @pl.when(g + 1 < _SC_CHG)
                    def _():
                        pltpu.async_copy(tab.at[sidx(g + 1)], rows[1 - bf],
                                         gsem[1 - bf])
                    pltpu.sync_copy(rows[bf], acc.at[didx(g)], add=True)
                return 0
            lax.fori_loop(0, _SC_CHG // 2, gbody, 0)

        def edges(tab):
            start_chunk_dma(0, 0)

            def ch2(m, _):
                chunk(tab, 2 * m, 0)
                chunk(tab, 2 * m + 1, 1)
                return 0
            lax.fori_loop(0, _SC_NCH // 2, ch2, 0)
            chunk(tab, _SC_NCH - 1, 0)

        def writeout(out_h):
            woff = 0
            while woff < _SC_WRT:
                sz = min(128, _SC_WRT - woff)
                pltpu.sync_copy(acc.at[pl.ds(s * _SC_WRT + woff, sz)],
                                rows[0].at[pl.ds(0, sz)])
                pltpu.sync_copy(rows[0].at[pl.ds(0, sz)],
                                out_h.at[pl.ds(s * _SC_WRT + woff, sz)])
                woff += sz

        for p in range(npass):
            def zbody(i, _):
                for j in range(COLW // 16):
                    rows[0][i, pl.ds(j * 16, 16)] = zeros16
                return 0
            lax.fori_loop(0, 128, zbody, 0)
            zoff = 0
            while zoff < _SC_WRT:
                sz = min(128, _SC_WRT - zoff)
                pltpu.sync_copy(rows[0].at[pl.ds(0, sz)],
                                acc.at[pl.ds(s * _SC_WRT + zoff, sz)])
                zoff += sz
            plsc.subcore_barrier()

            @pl.when(c == 0)
            def _():
                edges(tabs[2 * p])

            @pl.when(c == 1)
            def _():
                edges(tabs[2 * p + 1])
            plsc.subcore_barrier()

            @pl.when(c == 0)
            def _():
                writeout(outs[2 * p])

            @pl.when(c == 1)
            def _():
                writeout(outs[2 * p + 1])
            plsc.subcore_barrier()

    return edge_scatter


_edge_scatter2 = _make_edge_scatter(2)
_edge_scatter4 = _make_edge_scatter(4)


def _k0_body(f_ref, po0, po1, pi0, pi1, t0_ref, t1_ref, ns_ref, nd_ref):
    dout = po0[...] + po1[...]
    din = pi0[...] + pi1[...]
    ns = jnp.where(dout > 0, lax.rsqrt(jnp.maximum(dout, 1.0)), 0.0)
    nd = jnp.where(din > 0, lax.rsqrt(jnp.maximum(din, 1.0)), 0.0)
    ns_ref[...] = ns
    nd_ref[...] = nd
    t = f_ref[...] * ns
    for q, r in enumerate((t0_ref, t1_ref)):
        r[...] = t[:, q * COLW:(q + 1) * COLW]


_k0 = pl.pallas_call(
    _k0_body,
    grid=(NBLK,),
    in_specs=[
        pl.BlockSpec((BLK, INP), lambda i: (i, 0)),
        pl.BlockSpec((BLK, 1), lambda i: (i, 0)),
        pl.BlockSpec((BLK, 1), lambda i: (i, 0)),
        pl.BlockSpec((BLK, 1), lambda i: (i, 0)),
        pl.BlockSpec((BLK, 1), lambda i: (i, 0)),
    ],
    out_specs=[pl.BlockSpec((BLK, COLW), lambda i: (i, 0))] * 2
    + [
        pl.BlockSpec((BLK, 1), lambda i: (i, 0)),
        pl.BlockSpec((BLK, 1), lambda i: (i, 0)),
    ],
    out_shape=[jax.ShapeDtypeStruct((N, COLW), jnp.float32)] * 2
    + [
        jax.ShapeDtypeStruct((N, 1), jnp.float32),
        jax.ShapeDtypeStruct((N, 1), jnp.float32),
    ],
)


def _make_ka(nparts):
    Wd = nparts * COLW

    def body(*refs):
        us = refs[:nparts]
        nd_ref, w_ref, b_ref, z_ref, st_ref = refs[nparts:]
        u = jnp.concatenate([r[...] for r in us], axis=1)
        z = jnp.dot(u * nd_ref[...], w_ref[...],
                    preferred_element_type=jnp.float32,
                    precision=lax.Precision.HIGHEST) + b_ref[...]
        z_ref[...] = z

        @pl.when(pl.program_id(0) == 0)
        def _():
            st_ref[...] = jnp.zeros((1, H), jnp.float32)

        st_ref[...] += jnp.sum(z, 0, keepdims=True)

    return pl.pallas_call(
        body,
        grid=(NBLK,),
        in_specs=[pl.BlockSpec((BLK, COLW), lambda i: (i, 0))] * nparts
        + [
            pl.BlockSpec((BLK, 1), lambda i: (i, 0)),
            pl.BlockSpec((Wd, H), lambda i: (0, 0)),
            pl.BlockSpec((1, H), lambda i: (0, 0)),
        ],
        out_specs=[
            pl.BlockSpec((BLK, H), lambda i: (i, 0)),
            pl.BlockSpec((1, H), lambda i: (0, 0)),
        ],
        out_shape=[
            jax.ShapeDtypeStruct((N, H), jnp.float32),
            jax.ShapeDtypeStruct((1, H), jnp.float32),
        ],
    )


_ka64 = _make_ka(2)
_ka128 = _make_ka(4)


def _kv_body(z_ref, st_ref, vs_ref):
    mu = st_ref[...] * (1.0 / N)
    d = z_ref[...] - mu

    @pl.when(pl.program_id(0) == 0)
    def _():
        vs_ref[...] = jnp.zeros((1, H), jnp.float32)

    vs_ref[...] += jnp.sum(d * d, 0, keepdims=True)


_kv = pl.pallas_call(
    _kv_body,
    grid=(NBLK,),
    in_specs=[
        pl.BlockSpec((BLK, H), lambda i: (i, 0)),
        pl.BlockSpec((1, H), lambda i: (0, 0)),
    ],
    out_specs=pl.BlockSpec((1, H), lambda i: (0, 0)),
    out_shape=jax.ShapeDtypeStruct((1, H), jnp.float32),
)


def _bn_coeffs(st, vs, g, bb):
    mu = st * (1.0 / N)
    var = vs * (1.0 / N)
    a = g * lax.rsqrt(var + EPS)
    cc = bb - mu * a
    return a, cc


def _kb_mid_body(z_ref, st_ref, vs_ref, ns_ref, g_ref, bb_ref,
                 t0_ref, t1_ref, t2_ref, t3_ref):
    a, cc = _bn_coeffs(st_ref[...], vs_ref[...], g_ref[...], bb_ref[...])
    y = jnp.maximum(z_ref[...] * a + cc, 0.0)
    t = y * ns_ref[...]
    cw = H // 4
    for q, r in enumerate((t0_ref, t1_ref, t2_ref, t3_ref)):
        r[...] = t[:, q * cw:(q + 1) * cw]


_kb_mid = pl.pallas_call(
    _kb_mid_body,
    grid=(NBLK,),
    in_specs=[
        pl.BlockSpec((BLK, H), lambda i: (i, 0)),
        pl.BlockSpec((1, H), lambda i: (0, 0)),
        pl.BlockSpec((1, H), lambda i: (0, 0)),
        pl.BlockSpec((BLK, 1), lambda i: (i, 0)),
        pl.BlockSpec((1, H), lambda i: (0, 0)),
        pl.BlockSpec((1, H), lambda i: (0, 0)),
    ],
    out_specs=[pl.BlockSpec((BLK, H // 4), lambda i: (i, 0))] * 4,
    out_shape=[jax.ShapeDtypeStruct((N, H // 4), jnp.float32)] * 4,
)


def _kb_fin_body(z_ref, st_ref, vs_ref, g_ref, bb_ref, aww_ref, awb_ref,
                 gid_ref, aw_ref, seg_ref):
    a, cc = _bn_coeffs(st_ref[...], vs_ref[...], g_ref[...], bb_ref[...])
    y = jnp.maximum(z_ref[...] * a + cc, 0.0)
    aw = jnp.sum(y * aww_ref[...], axis=1, keepdims=True) + awb_ref[...]
    aw_ref[...] = aw
    w = 1.0 / (1.0 + jnp.exp(-aw))
    hw = y * w
    oh = (gid_ref[...] == lax.broadcasted_iota(jnp.int32, (BLK, G), 1)
          ).astype(jnp.float32)

    @pl.when(pl.program_id(0) == 0)
    def _():
        seg_ref[...] = jnp.zeros((G, H), jnp.float32)

    seg_ref[...] += lax.dot_general(oh, hw, (((0,), (0,)), ((), ())),
                                    preferred_element_type=jnp.float32,
                                    precision=lax.Precision.HIGHEST)


_kb_fin = pl.pallas_call(
    _kb_fin_body,
    grid=(NBLK,),
    in_specs=[
        pl.BlockSpec((BLK, H), lambda i: (i, 0)),
        pl.BlockSpec((1, H), lambda i: (0, 0)),
        pl.BlockSpec((1, H), lambda i: (0, 0)),
        pl.BlockSpec((1, H), lambda i: (0, 0)),
        pl.BlockSpec((1, H), lambda i: (0, 0)),
        pl.BlockSpec((1, H), lambda i: (0, 0)),
        pl.BlockSpec((1, 1), lambda i: (0, 0)),
        pl.BlockSpec((BLK, 1), lambda i: (i, 0)),
    ],
    out_specs=[
        pl.BlockSpec((BLK, 1), lambda i: (i, 0)),
        pl.BlockSpec((G, H), lambda i: (0, 0)),
    ],
    out_shape=[
        jax.ShapeDtypeStruct((N, 1), jnp.float32),
        jax.ShapeDtypeStruct((G, H), jnp.float32),
    ],
)


def _head_body(seg_ref, w1_ref, b1_ref, g1_ref, c1_ref, w2_ref, b2_ref,
               g2_ref, c2_ref, w3_ref, b3_ref, o_ref):
    x = jnp.dot(seg_ref[...], w1_ref[...], preferred_element_type=jnp.float32,
                precision=lax.Precision.HIGHEST) + b1_ref[...]
    mu = jnp.mean(x, 0, keepdims=True)
    d = x - mu
    var = jnp.mean(d * d, 0, keepdims=True)
    x = jnp.maximum(d * lax.rsqrt(var + EPS) * g1_ref[...] + c1_ref[...], 0.0)
    x = jnp.dot(x, w2_ref[...], preferred_element_type=jnp.float32,
                precision=lax.Precision.HIGHEST) + b2_ref[...]
    mu = jnp.mean(x, 0, keepdims=True)
    d = x - mu
    var = jnp.mean(d * d, 0, keepdims=True)
    x = jnp.maximum(d * lax.rsqrt(var + EPS) * g2_ref[...] + c2_ref[...], 0.0)
    x = jnp.dot(x, w3_ref[...], preferred_element_type=jnp.float32,
                precision=lax.Precision.HIGHEST) + b3_ref[...]
    o_ref[...] = 1.0 / (1.0 + jnp.exp(-x))


_head = pl.pallas_call(
    _head_body,
    out_shape=jax.ShapeDtypeStruct((G, H), jnp.float32),
)


def kernel(feats, edge_index, node_graph_ids,
           gcn1_W, gcn1_b, gcn1_bn_g, gcn1_bn_b,
           gcn2_0_W, gcn2_0_b, gcn2_0_bn_g, gcn2_0_bn_b,
           gcn2_1_W, gcn2_1_b, gcn2_1_bn_g, gcn2_1_bn_b,
           aw_W, aw_b, fc1_W, fc1_b, bn1_g, bn1_b,
           lin0_W, lin0_b, bnl0_g, bnl0_b, fc2_W, fc2_b):
    src = edge_index[0]
    dst = edge_index[1]
    feats64 = jnp.pad(feats, ((0, 0), (0, INP - IN)))
    W1p = jnp.pad(gcn1_W, ((0, INP - IN), (0, 0)))
    gids2 = node_graph_ids.reshape(N, 1)

    npad = EROWS * 128 - E
    pidx = jnp.arange(npad, dtype=jnp.int32)
    src2 = jnp.concatenate([src, pidx % 128])
    dst2 = jnp.concatenate([dst, N + pidx % 48])

    degflat = _deg_kernel(src, dst)
    degr = degflat.reshape(2, 2, DEGP)
    po0 = degr[0, 0, :N].reshape(N, 1)
    pi0 = degr[0, 1, :N].reshape(N, 1)
    po1 = degr[1, 0, :N].reshape(N, 1)
    pi1 = degr[1, 1, :N].reshape(N, 1)

    t0a, t0b, ns, nd = _k0(feats64, po0, po1, pi0, pi1)

    u1 = _edge_scatter2(src2, dst2, t0a, t0b)
    z1, st1 = _ka64(*u1, nd, W1p, gcn1_b.reshape(1, H))
    vs1 = _kv(z1, st1)
    t1 = _kb_mid(z1, st1, vs1, ns, gcn1_bn_g.reshape(1, H),
                 gcn1_bn_b.reshape(1, H))

    u2 = _edge_scatter4(src2, dst2, *t1)
    z2, st2 = _ka128(*u2, nd, gcn2_0_W, gcn2_0_b.reshape(1, H))
    vs2 = _kv(z2, st2)
    t2 = _kb_mid(z2, st2, vs2, ns, gcn2_0_bn_g.reshape(1, H),
                 gcn2_0_bn_b.reshape(1, H))

    u3 = _edge_scatter4(src2, dst2, *t2)
    z3, st3 = _ka128(*u3, nd, gcn2_1_W, gcn2_1_b.reshape(1, H))
    vs3 = _kv(z3, st3)
    aw, seg = _kb_fin(z3, st3, vs3, gcn2_1_bn_g.reshape(1, H),
                      gcn2_1_bn_b.reshape(1, H), aw_W.reshape(1, H),
                      aw_b.reshape(1, 1), gids2)

    w3p = jnp.pad(fc2_W, ((0, 0), (0, H - OUT)))
    b3p = jnp.pad(fc2_b, ((0, H - OUT))).reshape(1, H)
    headp = _head(seg, fc1_W, fc1_b.reshape(1, 256), bn1_g.reshape(1, 256),
                  bn1_b.reshape(1, 256), lin0_W, lin0_b.reshape(1, H),
                  bnl0_g.reshape(1, H), bnl0_b.reshape(1, H), w3p, b3p)
    x = headp[:, :OUT]
    return (x, aw)


# speedup vs baseline: 1.3022x; 1.3014x over previous
import functools

import jax
import jax.numpy as jnp
from jax import lax
from jax.experimental import pallas as pl
from jax.experimental.pallas import tpu as pltpu
from jax.experimental.pallas import tpu_sc as plsc

N = 50000
E = 1600000
G = 1024
IN = 38
INP = 64
H = 128
OUT = 67
EPS = 1e-5

BLK = 2000
NBLK = N // BLK

DEGP = 50048
DEGB = 2 * DEGP


def _vsmesh():
    return plsc.VectorSubcoreMesh(core_axis_name="c", subcore_axis_name="s")


_DEG_EPC = E // 2
_DEG_EPT = _DEG_EPC // 16
_DEG_CH = 4096
_DEG_NCH = _DEG_EPT // _DEG_CH
_DEG_TAIL = _DEG_EPT - _DEG_NCH * _DEG_CH
_DEG_TAILP = 896
_DEG_SL = DEGB // 16


@functools.partial(
    pl.kernel,
    out_type=jax.ShapeDtypeStruct((2 * DEGB,), jnp.float32),
    mesh=_vsmesh(),
    scratch_types=[
        pltpu.VMEM((_DEG_CH,), jnp.int32),
        pltpu.VMEM((128,), jnp.int32),
        pltpu.VMEM((128,), jnp.float32),
        pltpu.VMEM((_DEG_SL,), jnp.float32),
        pltpu.VMEM_SHARED((DEGB,), jnp.float32),
    ],
)
def _deg_kernel(src_h, dst_h, out_h, idxb, idxg, onesb, stage, acc):
    c = lax.axis_index("c")
    s = lax.axis_index("s")
    iota16 = lax.iota(jnp.int32, 16)
    ones16 = jnp.ones((16,), jnp.float32)
    zeros16 = jnp.zeros((16,), jnp.float32)
    for j in range(8):
        onesb[pl.ds(j * 16, 16)] = ones16

    def zbody(j, _):
        stage[pl.ds(j * 16, 16)] = zeros16
        return 0
    lax.fori_loop(0, _DEG_SL // 16, zbody, 0)
    pltpu.sync_copy(stage, acc.at[pl.ds(s * _DEG_SL, _DEG_SL)])
    plsc.subcore_barrier()
    ebase = c * _DEG_EPC + s * _DEG_EPT

    def do_groups(n_groups, off):
        def gbody(g, _):
            def cb(j, _2):
                v = idxb[pl.ds(g * 128 + j * 16, 16)]
                idxg[pl.ds(j * 16, 16)] = v + off
                return 0
            lax.fori_loop(0, 8, cb, 0)
            pltpu.sync_copy(onesb, acc.at[idxg], add=True)
            return 0
        lax.fori_loop(0, n_groups, gbody, 0)

    def chbody(ch, _):
        cb0 = ebase + ch * _DEG_CH
        pltpu.sync_copy(src_h.at[pl.ds(cb0, _DEG_CH)], idxb)
        do_groups(_DEG_CH // 128, 0)
        pltpu.sync_copy(dst_h.at[pl.ds(cb0, _DEG_CH)], idxb)
        do_groups(_DEG_CH // 128, DEGP)
        return 0
    lax.fori_loop(0, _DEG_NCH, chbody, 0)

    tb = ebase + _DEG_NCH * _DEG_CH
    for arr_h, off in ((src_h, 0), (dst_h, DEGP)):
        pltpu.sync_copy(arr_h.at[pl.ds(tb, _DEG_TAIL)],
                        idxb.at[pl.ds(0, _DEG_TAIL)])
        for j in range((_DEG_TAILP - _DEG_TAIL) // 16):
            idxb[pl.ds(_DEG_TAIL + j * 16, 16)] = N + iota16
        do_groups(_DEG_TAILP // 128, off)

    plsc.subcore_barrier()
    pltpu.sync_copy(acc.at[pl.ds(s * _DEG_SL, _DEG_SL)], stage)
    pltpu.sync_copy(stage, out_h.at[pl.ds(c * DEGB + s * _DEG_SL, _DEG_SL)])


EROWS = 12800
_SC_RPT = EROWS // 16
_SC_CHG = 32
_SC_NCH = _SC_RPT // _SC_CHG
NP2 = 50048
_SC_WRT = NP2 // 16
COLW = 32


def _make_edge_scatter(NBLKT):
    npass = NBLKT // 2

    @functools.partial(
        pl.kernel,
        out_type=[jax.ShapeDtypeStruct((NP2, COLW), jnp.float32)
                  for _ in range(NBLKT)],
        mesh=_vsmesh(),
        scratch_types=(
            [pltpu.VMEM_SHARED((NP2, COLW), jnp.float32)]
            + [pltpu.VMEM((_SC_CHG * 128,), jnp.int32) for _ in range(2)]
            + [pltpu.VMEM((_SC_CHG * 128,), jnp.int32) for _ in range(2)]
            + [pltpu.VMEM((128, COLW), jnp.float32) for _ in range(2)]
            + [pltpu.SemaphoreType.DMA for _ in range(3)]
        ),
        compiler_params=pltpu.CompilerParams(use_tc_tiling_on_sc=False),
    )
    def edge_scatter(src_h, dst_h, *rest):
        tabs = rest[:NBLKT]
        outs = rest[NBLKT:2 * NBLKT]
        sc = rest[2 * NBLKT:]
        acc = sc[0]
        srcb = sc[1:3]
        dstb = sc[3:5]
        rows = sc[5:7]
        csem = sc[7]
        gsem = sc[8:10]
        c = lax.axis_index("c")
        s = lax.axis_index("s")
        zeros16 = jnp.zeros((16,), jnp.float32)
        rbase = s * _SC_RPT

        CE = _SC_CHG * 128

        def start_chunk_dma(ch, b):
            e0 = (rbase + ch * _SC_CHG) * 128
            pltpu.async_copy(src_h.at[pl.ds(e0, CE)], srcb[b], csem)
            pltpu.async_copy(dst_h.at[pl.ds(e0, CE)], dstb[b], csem)

        def wait_chunk_dma(b):
            pltpu.make_async_copy(src_h.at[pl.ds(0, CE)], srcb[b],
                                  csem).wait()
            pltpu.make_async_copy(dst_h.at[pl.ds(0, CE)], dstb[b],
                                  csem).wait()

        def chunk(tab, ch, b):
            wait_chunk_dma(b)

            @pl.when(ch + 1 < _SC_NCH)
            def _():
                start_chunk_dma(ch + 1, 1 - b)

            sb, db = srcb[b], dstb[b]

            def sidx(g):
                return sb.at[pl.ds(g * 128, 128)]

            def didx(g):
                return db.at[pl.ds(g * 128, 128)]

            pltpu.async_copy(tab.at[sidx(0)], rows[0], gsem[0])

            def gbody(k, _):
                for j in range(2):
                    g = 2 * k + j
                    bf = j

                    @pl.when(g + 1 < _SC_CHG)
                    def _():
                        pltpu.async_copy(tab.at[sidx(g + 1)], rows[1 - bf],
                                         gsem[1 - bf])
                    pltpu.make_async_copy(tab.at[sidx(g)], rows[bf],
                                          gsem[bf]).wait()
                    pltpu.sync_copy(rows[bf], acc.at[didx(g)], add=True)
                return 0
            lax.fori_loop(0, _SC_CHG // 2, gbody, 0)

        def edges(tab):
            start_chunk_dma(0, 0)

            def ch2(m, _):
                chunk(tab, 2 * m, 0)
                chunk(tab, 2 * m + 1, 1)
                return 0
            lax.fori_loop(0, _SC_NCH // 2, ch2, 0)
            chunk(tab, _SC_NCH - 1, 0)

        def writeout(out_h):
            woff = 0
            while woff < _SC_WRT:
                sz = min(128, _SC_WRT - woff)
                pltpu.sync_copy(acc.at[pl.ds(s * _SC_WRT + woff, sz)],
                                rows[0].at[pl.ds(0, sz)])
                pltpu.sync_copy(rows[0].at[pl.ds(0, sz)],
                                out_h.at[pl.ds(s * _SC_WRT + woff, sz)])
                woff += sz

        for p in range(npass):
            def zbody(i, _):
                for j in range(COLW // 16):
                    rows[0][i, pl.ds(j * 16, 16)] = zeros16
                return 0
            lax.fori_loop(0, 128, zbody, 0)
            zoff = 0
            while zoff < _SC_WRT:
                sz = min(128, _SC_WRT - zoff)
                pltpu.sync_copy(rows[0].at[pl.ds(0, sz)],
                                acc.at[pl.ds(s * _SC_WRT + zoff, sz)])
                zoff += sz
            plsc.subcore_barrier()

            @pl.when(c == 0)
            def _():
                edges(tabs[2 * p])

            @pl.when(c == 1)
            def _():
                edges(tabs[2 * p + 1])
            plsc.subcore_barrier()

            @pl.when(c == 0)
            def _():
                writeout(outs[2 * p])

            @pl.when(c == 1)
            def _():
                writeout(outs[2 * p + 1])
            plsc.subcore_barrier()

    return edge_scatter


_edge_scatter2 = _make_edge_scatter(2)
_edge_scatter4 = _make_edge_scatter(4)


def _k0_body(f_ref, po0, po1, pi0, pi1, t0_ref, t1_ref, ns_ref, nd_ref):
    dout = po0[...] + po1[...]
    din = pi0[...] + pi1[...]
    ns = jnp.where(dout > 0, lax.rsqrt(jnp.maximum(dout, 1.0)), 0.0)
    nd = jnp.where(din > 0, lax.rsqrt(jnp.maximum(din, 1.0)), 0.0)
    ns_ref[...] = ns
    nd_ref[...] = nd
    t = f_ref[...] * ns
    for q, r in enumerate((t0_ref, t1_ref)):
        r[...] = t[:, q * COLW:(q + 1) * COLW]


_k0 = pl.pallas_call(
    _k0_body,
    grid=(NBLK,),
    in_specs=[
        pl.BlockSpec((BLK, INP), lambda i: (i, 0)),
        pl.BlockSpec((BLK, 1), lambda i: (i, 0)),
        pl.BlockSpec((BLK, 1), lambda i: (i, 0)),
        pl.BlockSpec((BLK, 1), lambda i: (i, 0)),
        pl.BlockSpec((BLK, 1), lambda i: (i, 0)),
    ],
    out_specs=[pl.BlockSpec((BLK, COLW), lambda i: (i, 0))] * 2
    + [
        pl.BlockSpec((BLK, 1), lambda i: (i, 0)),
        pl.BlockSpec((BLK, 1), lambda i: (i, 0)),
    ],
    out_shape=[jax.ShapeDtypeStruct((N, COLW), jnp.float32)] * 2
    + [
        jax.ShapeDtypeStruct((N, 1), jnp.float32),
        jax.ShapeDtypeStruct((N, 1), jnp.float32),
    ],
)


def _make_ka(nparts):
    Wd = nparts * COLW

    def body(*refs):
        us = refs[:nparts]
        nd_ref, w_ref, b_ref, z_ref, st_ref = refs[nparts:]
        u = jnp.concatenate([r[...] for r in us], axis=1)
        z = jnp.dot(u * nd_ref[...], w_ref[...],
                    preferred_element_type=jnp.float32,
                    precision=lax.Precision.HIGHEST) + b_ref[...]
        z_ref[...] = z

        @pl.when(pl.program_id(0) == 0)
        def _():
            st_ref[...] = jnp.zeros((1, H), jnp.float32)

        st_ref[...] += jnp.sum(z, 0, keepdims=True)

    return pl.pallas_call(
        body,
        grid=(NBLK,),
        in_specs=[pl.BlockSpec((BLK, COLW), lambda i: (i, 0))] * nparts
        + [
            pl.BlockSpec((BLK, 1), lambda i: (i, 0)),
            pl.BlockSpec((Wd, H), lambda i: (0, 0)),
            pl.BlockSpec((1, H), lambda i: (0, 0)),
        ],
        out_specs=[
            pl.BlockSpec((BLK, H), lambda i: (i, 0)),
            pl.BlockSpec((1, H), lambda i: (0, 0)),
        ],
        out_shape=[
            jax.ShapeDtypeStruct((N, H), jnp.float32),
            jax.ShapeDtypeStruct((1, H), jnp.float32),
        ],
    )


_ka64 = _make_ka(2)
_ka128 = _make_ka(4)


def _kv_body(z_ref, st_ref, vs_ref):
    mu = st_ref[...] * (1.0 / N)
    d = z_ref[...] - mu

    @pl.when(pl.program_id(0) == 0)
    def _():
        vs_ref[...] = jnp.zeros((1, H), jnp.float32)

    vs_ref[...] += jnp.sum(d * d, 0, keepdims=True)


_kv = pl.pallas_call(
    _kv_body,
    grid=(NBLK,),
    in_specs=[
        pl.BlockSpec((BLK, H), lambda i: (i, 0)),
        pl.BlockSpec((1, H), lambda i: (0, 0)),
    ],
    out_specs=pl.BlockSpec((1, H), lambda i: (0, 0)),
    out_shape=jax.ShapeDtypeStruct((1, H), jnp.float32),
)


def _bn_coeffs(st, vs, g, bb):
    mu = st * (1.0 / N)
    var = vs * (1.0 / N)
    a = g * lax.rsqrt(var + EPS)
    cc = bb - mu * a
    return a, cc


def _kb_mid_body(z_ref, st_ref, vs_ref, ns_ref, g_ref, bb_ref,
                 t0_ref, t1_ref, t2_ref, t3_ref):
    a, cc = _bn_coeffs(st_ref[...], vs_ref[...], g_ref[...], bb_ref[...])
    y = jnp.maximum(z_ref[...] * a + cc, 0.0)
    t = y * ns_ref[...]
    cw = H // 4
    for q, r in enumerate((t0_ref, t1_ref, t2_ref, t3_ref)):
        r[...] = t[:, q * cw:(q + 1) * cw]


_kb_mid = pl.pallas_call(
    _kb_mid_body,
    grid=(NBLK,),
    in_specs=[
        pl.BlockSpec((BLK, H), lambda i: (i, 0)),
        pl.BlockSpec((1, H), lambda i: (0, 0)),
        pl.BlockSpec((1, H), lambda i: (0, 0)),
        pl.BlockSpec((BLK, 1), lambda i: (i, 0)),
        pl.BlockSpec((1, H), lambda i: (0, 0)),
        pl.BlockSpec((1, H), lambda i: (0, 0)),
    ],
    out_specs=[pl.BlockSpec((BLK, H // 4), lambda i: (i, 0))] * 4,
    out_shape=[jax.ShapeDtypeStruct((N, H // 4), jnp.float32)] * 4,
)


def _kb_fin_body(z_ref, st_ref, vs_ref, g_ref, bb_ref, aww_ref, awb_ref,
                 gid_ref, aw_ref, seg_ref):
    a, cc = _bn_coeffs(st_ref[...], vs_ref[...], g_ref[...], bb_ref[...])
    y = jnp.maximum(z_ref[...] * a + cc, 0.0)
    aw = jnp.sum(y * aww_ref[...], axis=1, keepdims=True) + awb_ref[...]
    aw_ref[...] = aw
    w = 1.0 / (1.0 + jnp.exp(-aw))
    hw = y * w
    oh = (gid_ref[...] == lax.broadcasted_iota(jnp.int32, (BLK, G), 1)
          ).astype(jnp.float32)

    @pl.when(pl.program_id(0) == 0)
    def _():
        seg_ref[...] = jnp.zeros((G, H), jnp.float32)

    seg_ref[...] += lax.dot_general(oh, hw, (((0,), (0,)), ((), ())),
                                    preferred_element_type=jnp.float32,
                                    precision=lax.Precision.HIGHEST)


_kb_fin = pl.pallas_call(
    _kb_fin_body,
    grid=(NBLK,),
    in_specs=[
        pl.BlockSpec((BLK, H), lambda i: (i, 0)),
        pl.BlockSpec((1, H), lambda i: (0, 0)),
        pl.BlockSpec((1, H), lambda i: (0, 0)),
        pl.BlockSpec((1, H), lambda i: (0, 0)),
        pl.BlockSpec((1, H), lambda i: (0, 0)),
        pl.BlockSpec((1, H), lambda i: (0, 0)),
        pl.BlockSpec((1, 1), lambda i: (0, 0)),
        pl.BlockSpec((BLK, 1), lambda i: (i, 0)),
    ],
    out_specs=[
        pl.BlockSpec((BLK, 1), lambda i: (i, 0)),
        pl.BlockSpec((G, H), lambda i: (0, 0)),
    ],
    out_shape=[
        jax.ShapeDtypeStruct((N, 1), jnp.float32),
        jax.ShapeDtypeStruct((G, H), jnp.float32),
    ],
)


def _head_body(seg_ref, w1_ref, b1_ref, g1_ref, c1_ref, w2_ref, b2_ref,
               g2_ref, c2_ref, w3_ref, b3_ref, o_ref):
    x = jnp.dot(seg_ref[...], w1_ref[...], preferred_element_type=jnp.float32,
                precision=lax.Precision.HIGHEST) + b1_ref[...]
    mu = jnp.mean(x, 0, keepdims=True)
    d = x - mu
    var = jnp.mean(d * d, 0, keepdims=True)
    x = jnp.maximum(d * lax.rsqrt(var + EPS) * g1_ref[...] + c1_ref[...], 0.0)
    x = jnp.dot(x, w2_ref[...], preferred_element_type=jnp.float32,
                precision=lax.Precision.HIGHEST) + b2_ref[...]
    mu = jnp.mean(x, 0, keepdims=True)
    d = x - mu
    var = jnp.mean(d * d, 0, keepdims=True)
    x = jnp.maximum(d * lax.rsqrt(var + EPS) * g2_ref[...] + c2_ref[...], 0.0)
    x = jnp.dot(x, w3_ref[...], preferred_element_type=jnp.float32,
                precision=lax.Precision.HIGHEST) + b3_ref[...]
    o_ref[...] = 1.0 / (1.0 + jnp.exp(-x))


_head = pl.pallas_call(
    _head_body,
    out_shape=jax.ShapeDtypeStruct((G, H), jnp.float32),
)


def kernel(feats, edge_index, node_graph_ids,
           gcn1_W, gcn1_b, gcn1_bn_g, gcn1_bn_b,
           gcn2_0_W, gcn2_0_b, gcn2_0_bn_g, gcn2_0_bn_b,
           gcn2_1_W, gcn2_1_b, gcn2_1_bn_g, gcn2_1_bn_b,
           aw_W, aw_b, fc1_W, fc1_b, bn1_g, bn1_b,
           lin0_W, lin0_b, bnl0_g, bnl0_b, fc2_W, fc2_b):
    src = edge_index[0]
    dst = edge_index[1]
    feats64 = jnp.pad(feats, ((0, 0), (0, INP - IN)))
    W1p = jnp.pad(gcn1_W, ((0, INP - IN), (0, 0)))
    gids2 = node_graph_ids.reshape(N, 1)

    npad = EROWS * 128 - E
    pidx = jnp.arange(npad, dtype=jnp.int32)
    src2 = jnp.concatenate([src, pidx % 128])
    dst2 = jnp.concatenate([dst, N + pidx % 48])

    degflat = _deg_kernel(src, dst)
    degr = degflat.reshape(2, 2, DEGP)
    po0 = degr[0, 0, :N].reshape(N, 1)
    pi0 = degr[0, 1, :N].reshape(N, 1)
    po1 = degr[1, 0, :N].reshape(N, 1)
    pi1 = degr[1, 1, :N].reshape(N, 1)

    t0a, t0b, ns, nd = _k0(feats64, po0, po1, pi0, pi1)

    u1 = _edge_scatter2(src2, dst2, t0a, t0b)
    z1, st1 = _ka64(*u1, nd, W1p, gcn1_b.reshape(1, H))
    vs1 = _kv(z1, st1)
    t1 = _kb_mid(z1, st1, vs1, ns, gcn1_bn_g.reshape(1, H),
                 gcn1_bn_b.reshape(1, H))

    u2 = _edge_scatter4(src2, dst2, *t1)
    z2, st2 = _ka128(*u2, nd, gcn2_0_W, gcn2_0_b.reshape(1, H))
    vs2 = _kv(z2, st2)
    t2 = _kb_mid(z2, st2, vs2, ns, gcn2_0_bn_g.reshape(1, H),
                 gcn2_0_bn_b.reshape(1, H))

    u3 = _edge_scatter4(src2, dst2, *t2)
    z3, st3 = _ka128(*u3, nd, gcn2_1_W, gcn2_1_b.reshape(1, H))
    vs3 = _kv(z3, st3)
    aw, seg = _kb_fin(z3, st3, vs3, gcn2_1_bn_g.reshape(1, H),
                      gcn2_1_bn_b.reshape(1, H), aw_W.reshape(1, H),
                      aw_b.reshape(1, 1), gids2)

    w3p = jnp.pad(fc2_W, ((0, 0), (0, H - OUT)))
    b3p = jnp.pad(fc2_b, ((0, H - OUT))).reshape(1, H)
    headp = _head(seg, fc1_W, fc1_b.reshape(1, 256), bn1_g.reshape(1, 256),
                  bn1_b.reshape(1, 256), lin0_W, lin0_b.reshape(1, H),
                  bnl0_g.reshape(1, H), bnl0_b.reshape(1, H), w3p, b3p)
    x = headp[:, :OUT]
    return (x, aw)


# speedup vs baseline: 1.5408x; 1.1832x over previous
import functools

import jax
import jax.numpy as jnp
from jax import lax
from jax.experimental import pallas as pl
from jax.experimental.pallas import tpu as pltpu
from jax.experimental.pallas import tpu_sc as plsc

N = 50000
E = 1600000
G = 1024
IN = 38
INP = 64
H = 128
OUT = 67
EPS = 1e-5

BLK = 2000
NBLK = N // BLK

DEGP = 50048
DEGB = 2 * DEGP


def _vsmesh():
    return plsc.VectorSubcoreMesh(core_axis_name="c", subcore_axis_name="s")


_DEG_EPC = E // 2
_DEG_EPT = _DEG_EPC // 16
_DEG_CH = 4096
_DEG_NCH = _DEG_EPT // _DEG_CH
_DEG_TAIL = _DEG_EPT - _DEG_NCH * _DEG_CH
_DEG_TAILP = 896
_DEG_SL = DEGB // 16


@functools.partial(
    pl.kernel,
    out_type=jax.ShapeDtypeStruct((2 * DEGB,), jnp.float32),
    mesh=_vsmesh(),
    scratch_types=[
        pltpu.VMEM((_DEG_CH,), jnp.int32),
        pltpu.VMEM((128,), jnp.int32),
        pltpu.VMEM((128,), jnp.float32),
        pltpu.VMEM((_DEG_SL,), jnp.float32),
        pltpu.VMEM_SHARED((DEGB,), jnp.float32),
    ],
)
def _deg_kernel(src_h, dst_h, out_h, idxb, idxg, onesb, stage, acc):
    c = lax.axis_index("c")
    s = lax.axis_index("s")
    iota16 = lax.iota(jnp.int32, 16)
    ones16 = jnp.ones((16,), jnp.float32)
    zeros16 = jnp.zeros((16,), jnp.float32)
    for j in range(8):
        onesb[pl.ds(j * 16, 16)] = ones16

    def zbody(j, _):
        stage[pl.ds(j * 16, 16)] = zeros16
        return 0
    lax.fori_loop(0, _DEG_SL // 16, zbody, 0)
    pltpu.sync_copy(stage, acc.at[pl.ds(s * _DEG_SL, _DEG_SL)])
    plsc.subcore_barrier()
    ebase = c * _DEG_EPC + s * _DEG_EPT

    def do_groups(n_groups, off):
        def gbody(g, _):
            def cb(j, _2):
                v = idxb[pl.ds(g * 128 + j * 16, 16)]
                idxg[pl.ds(j * 16, 16)] = v + off
                return 0
            lax.fori_loop(0, 8, cb, 0)
            pltpu.sync_copy(onesb, acc.at[idxg], add=True)
            return 0
        lax.fori_loop(0, n_groups, gbody, 0)

    def chbody(ch, _):
        cb0 = ebase + ch * _DEG_CH
        pltpu.sync_copy(src_h.at[pl.ds(cb0, _DEG_CH)], idxb)
        do_groups(_DEG_CH // 128, 0)
        pltpu.sync_copy(dst_h.at[pl.ds(cb0, _DEG_CH)], idxb)
        do_groups(_DEG_CH // 128, DEGP)
        return 0
    lax.fori_loop(0, _DEG_NCH, chbody, 0)

    tb = ebase + _DEG_NCH * _DEG_CH
    for arr_h, off in ((src_h, 0), (dst_h, DEGP)):
        pltpu.sync_copy(arr_h.at[pl.ds(tb, _DEG_TAIL)],
                        idxb.at[pl.ds(0, _DEG_TAIL)])
        for j in range((_DEG_TAILP - _DEG_TAIL) // 16):
            idxb[pl.ds(_DEG_TAIL + j * 16, 16)] = N + iota16
        do_groups(_DEG_TAILP // 128, off)

    plsc.subcore_barrier()
    pltpu.sync_copy(acc.at[pl.ds(s * _DEG_SL, _DEG_SL)], stage)
    pltpu.sync_copy(stage, out_h.at[pl.ds(c * DEGB + s * _DEG_SL, _DEG_SL)])


EROWS = 12960
_SC_RPT = EROWS // 16
_SC_CHG = 30
_SC_NCH = _SC_RPT // _SC_CHG
NP2 = 50048
_SC_WRT = NP2 // 16
COLW = 32


def _make_edge_scatter(NBLKT):
    npass = NBLKT // 2

    @functools.partial(
        pl.kernel,
        out_type=[jax.ShapeDtypeStruct((NP2, COLW), jnp.float32)
                  for _ in range(NBLKT)],
        mesh=_vsmesh(),
        scratch_types=(
            [pltpu.VMEM_SHARED((NP2, COLW), jnp.float32)]
            + [pltpu.VMEM((_SC_CHG * 256,), jnp.int32) for _ in range(2)]
            + [pltpu.VMEM((128, COLW), jnp.float32) for _ in range(3)]
            + [pltpu.SemaphoreType.DMA for _ in range(4)]
        ),
        compiler_params=pltpu.CompilerParams(use_tc_tiling_on_sc=False),
    )
    def edge_scatter(eidx_h, *rest):
        tabs = rest[:NBLKT]
        outs = rest[NBLKT:2 * NBLKT]
        sc = rest[2 * NBLKT:]
        acc = sc[0]
        ebuf = sc[1:3]
        rows = sc[3:6]
        csem = sc[6]
        gsem = sc[7:10]
        c = lax.axis_index("c")
        s = lax.axis_index("s")
        zeros16 = jnp.zeros((16,), jnp.float32)
        rbase = s * _SC_RPT

        CE = _SC_CHG * 256

        def start_chunk_dma(ch, b):
            e0 = (rbase + ch * _SC_CHG) * 256
            pltpu.async_copy(eidx_h.at[pl.ds(e0, CE)], ebuf[b], csem)

        def wait_chunk_dma(b):
            pltpu.make_async_copy(eidx_h.at[pl.ds(0, CE)], ebuf[b],
                                  csem).wait()

        def chunk(tab, ch, b):
            wait_chunk_dma(b)

            @pl.when(ch + 1 < _SC_NCH)
            def _():
                start_chunk_dma(ch + 1, 1 - b)

            eb = ebuf[b]

            def sidx(g):
                return eb.at[pl.ds(g * 256, 128)]

            def didx(g):
                return eb.at[pl.ds(g * 256 + 128, 128)]

            for g0 in range(2):
                pltpu.async_copy(tab.at[sidx(g0)], rows[g0], gsem[g0])

            def gbody(k, _):
                for j in range(3):
                    g = 3 * k + j
                    bf = j
                    nb = (j + 2) % 3

                    @pl.when(g + 2 < _SC_CHG)
                    def _():
                        pltpu.async_copy(tab.at[sidx(g + 2)], rows[nb],
                                         gsem[nb])
                    pltpu.make_async_copy(tab.at[sidx(g)], rows[bf],
                                          gsem[bf]).wait()
                    pltpu.sync_copy(rows[bf], acc.at[didx(g)], add=True)
                return 0
            lax.fori_loop(0, _SC_CHG // 3, gbody, 0)

        def edges(tab):
            start_chunk_dma(0, 0)

            def ch2(m, _):
                chunk(tab, 2 * m, 0)
                chunk(tab, 2 * m + 1, 1)
                return 0
            lax.fori_loop(0, _SC_NCH // 2, ch2, 0)
            chunk(tab, _SC_NCH - 1, 0)

        def writeout(out_h):
            woff = 0
            while woff < _SC_WRT:
                sz = min(128, _SC_WRT - woff)
                pltpu.sync_copy(acc.at[pl.ds(s * _SC_WRT + woff, sz)],
                                rows[0].at[pl.ds(0, sz)])
                pltpu.sync_copy(rows[0].at[pl.ds(0, sz)],
                                out_h.at[pl.ds(s * _SC_WRT + woff, sz)])
                woff += sz

        for p in range(npass):
            def zbody(i, _):
                for j in range(COLW // 16):
                    rows[0][i, pl.ds(j * 16, 16)] = zeros16
                return 0
            lax.fori_loop(0, 128, zbody, 0)
            zoff = 0
            while zoff < _SC_WRT:
                sz = min(128, _SC_WRT - zoff)
                pltpu.sync_copy(rows[0].at[pl.ds(0, sz)],
                                acc.at[pl.ds(s * _SC_WRT + zoff, sz)])
                zoff += sz
            plsc.subcore_barrier()

            @pl.when(c == 0)
            def _():
                edges(tabs[2 * p])

            @pl.when(c == 1)
            def _():
                edges(tabs[2 * p + 1])
            plsc.subcore_barrier()

            @pl.when(c == 0)
            def _():
                writeout(outs[2 * p])

            @pl.when(c == 1)
            def _():
                writeout(outs[2 * p + 1])
            plsc.subcore_barrier()

    return edge_scatter


_edge_scatter2 = _make_edge_scatter(2)
_edge_scatter4 = _make_edge_scatter(4)


def _k0_body(f_ref, po0, po1, pi0, pi1, t0_ref, t1_ref, ns_ref, nd_ref):
    dout = po0[...] + po1[...]
    din = pi0[...] + pi1[...]
    ns = jnp.where(dout > 0, lax.rsqrt(jnp.maximum(dout, 1.0)), 0.0)
    nd = jnp.where(din > 0, lax.rsqrt(jnp.maximum(din, 1.0)), 0.0)
    ns_ref[...] = ns
    nd_ref[...] = nd
    t = f_ref[...] * ns
    for q, r in enumerate((t0_ref, t1_ref)):
        r[...] = t[:, q * COLW:(q + 1) * COLW]


_k0 = pl.pallas_call(
    _k0_body,
    grid=(NBLK,),
    in_specs=[
        pl.BlockSpec((BLK, INP), lambda i: (i, 0)),
        pl.BlockSpec((BLK, 1), lambda i: (i, 0)),
        pl.BlockSpec((BLK, 1), lambda i: (i, 0)),
        pl.BlockSpec((BLK, 1), lambda i: (i, 0)),
        pl.BlockSpec((BLK, 1), lambda i: (i, 0)),
    ],
    out_specs=[pl.BlockSpec((BLK, COLW), lambda i: (i, 0))] * 2
    + [
        pl.BlockSpec((BLK, 1), lambda i: (i, 0)),
        pl.BlockSpec((BLK, 1), lambda i: (i, 0)),
    ],
    out_shape=[jax.ShapeDtypeStruct((N, COLW), jnp.float32)] * 2
    + [
        jax.ShapeDtypeStruct((N, 1), jnp.float32),
        jax.ShapeDtypeStruct((N, 1), jnp.float32),
    ],
)


def _make_ka(nparts):
    Wd = nparts * COLW

    def body(*refs):
        us = refs[:nparts]
        nd_ref, w_ref, b_ref, z_ref, st_ref = refs[nparts:]
        u = jnp.concatenate([r[...] for r in us], axis=1)
        z = jnp.dot(u * nd_ref[...], w_ref[...],
                    preferred_element_type=jnp.float32,
                    precision=lax.Precision.HIGHEST) + b_ref[...]
        z_ref[...] = z

        @pl.when(pl.program_id(0) == 0)
        def _():
            st_ref[...] = jnp.zeros((1, H), jnp.float32)

        st_ref[...] += jnp.sum(z, 0, keepdims=True)

    return pl.pallas_call(
        body,
        grid=(NBLK,),
        in_specs=[pl.BlockSpec((BLK, COLW), lambda i: (i, 0))] * nparts
        + [
            pl.BlockSpec((BLK, 1), lambda i: (i, 0)),
            pl.BlockSpec((Wd, H), lambda i: (0, 0)),
            pl.BlockSpec((1, H), lambda i: (0, 0)),
        ],
        out_specs=[
            pl.BlockSpec((BLK, H), lambda i: (i, 0)),
            pl.BlockSpec((1, H), lambda i: (0, 0)),
        ],
        out_shape=[
            jax.ShapeDtypeStruct((N, H), jnp.float32),
            jax.ShapeDtypeStruct((1, H), jnp.float32),
        ],
    )


_ka64 = _make_ka(2)
_ka128 = _make_ka(4)


def _kv_body(z_ref, st_ref, vs_ref):
    mu = st_ref[...] * (1.0 / N)
    d = z_ref[...] - mu

    @pl.when(pl.program_id(0) == 0)
    def _():
        vs_ref[...] = jnp.zeros((1, H), jnp.float32)

    vs_ref[...] += jnp.sum(d * d, 0, keepdims=True)


_kv = pl.pallas_call(
    _kv_body,
    grid=(NBLK,),
    in_specs=[
        pl.BlockSpec((BLK, H), lambda i: (i, 0)),
        pl.BlockSpec((1, H), lambda i: (0, 0)),
    ],
    out_specs=pl.BlockSpec((1, H), lambda i: (0, 0)),
    out_shape=jax.ShapeDtypeStruct((1, H), jnp.float32),
)


def _bn_coeffs(st, vs, g, bb):
    mu = st * (1.0 / N)
    var = vs * (1.0 / N)
    a = g * lax.rsqrt(var + EPS)
    cc = bb - mu * a
    return a, cc


def _kb_mid_body(z_ref, st_ref, vs_ref, ns_ref, g_ref, bb_ref,
                 t0_ref, t1_ref, t2_ref, t3_ref):
    a, cc = _bn_coeffs(st_ref[...], vs_ref[...], g_ref[...], bb_ref[...])
    y = jnp.maximum(z_ref[...] * a + cc, 0.0)
    t = y * ns_ref[...]
    cw = H // 4
    for q, r in enumerate((t0_ref, t1_ref, t2_ref, t3_ref)):
        r[...] = t[:, q * cw:(q + 1) * cw]


_kb_mid = pl.pallas_call(
    _kb_mid_body,
    grid=(NBLK,),
    in_specs=[
        pl.BlockSpec((BLK, H), lambda i: (i, 0)),
        pl.BlockSpec((1, H), lambda i: (0, 0)),
        pl.BlockSpec((1, H), lambda i: (0, 0)),
        pl.BlockSpec((BLK, 1), lambda i: (i, 0)),
        pl.BlockSpec((1, H), lambda i: (0, 0)),
        pl.BlockSpec((1, H), lambda i: (0, 0)),
    ],
    out_specs=[pl.BlockSpec((BLK, H // 4), lambda i: (i, 0))] * 4,
    out_shape=[jax.ShapeDtypeStruct((N, H // 4), jnp.float32)] * 4,
)


def _kb_fin_body(z_ref, st_ref, vs_ref, g_ref, bb_ref, aww_ref, awb_ref,
                 gid_ref, aw_ref, seg_ref):
    a, cc = _bn_coeffs(st_ref[...], vs_ref[...], g_ref[...], bb_ref[...])
    y = jnp.maximum(z_ref[...] * a + cc, 0.0)
    aw = jnp.sum(y * aww_ref[...], axis=1, keepdims=True) + awb_ref[...]
    aw_ref[...] = aw
    w = 1.0 / (1.0 + jnp.exp(-aw))
    hw = y * w
    oh = (gid_ref[...] == lax.broadcasted_iota(jnp.int32, (BLK, G), 1)
          ).astype(jnp.float32)

    @pl.when(pl.program_id(0) == 0)
    def _():
        seg_ref[...] = jnp.zeros((G, H), jnp.float32)

    seg_ref[...] += lax.dot_general(oh, hw, (((0,), (0,)), ((), ())),
                                    preferred_element_type=jnp.float32,
                                    precision=lax.Precision.HIGHEST)


_kb_fin = pl.pallas_call(
    _kb_fin_body,
    grid=(NBLK,),
    in_specs=[
        pl.BlockSpec((BLK, H), lambda i: (i, 0)),
        pl.BlockSpec((1, H), lambda i: (0, 0)),
        pl.BlockSpec((1, H), lambda i: (0, 0)),
        pl.BlockSpec((1, H), lambda i: (0, 0)),
        pl.BlockSpec((1, H), lambda i: (0, 0)),
        pl.BlockSpec((1, H), lambda i: (0, 0)),
        pl.BlockSpec((1, 1), lambda i: (0, 0)),
        pl.BlockSpec((BLK, 1), lambda i: (i, 0)),
    ],
    out_specs=[
        pl.BlockSpec((BLK, 1), lambda i: (i, 0)),
        pl.BlockSpec((G, H), lambda i: (0, 0)),
    ],
    out_shape=[
        jax.ShapeDtypeStruct((N, 1), jnp.float32),
        jax.ShapeDtypeStruct((G, H), jnp.float32),
    ],
)


def _head_body(seg_ref, w1_ref, b1_ref, g1_ref, c1_ref, w2_ref, b2_ref,
               g2_ref, c2_ref, w3_ref, b3_ref, o_ref):
    x = jnp.dot(seg_ref[...], w1_ref[...], preferred_element_type=jnp.float32,
                precision=lax.Precision.HIGHEST) + b1_ref[...]
    mu = jnp.mean(x, 0, keepdims=True)
    d = x - mu
    var = jnp.mean(d * d, 0, keepdims=True)
    x = jnp.maximum(d * lax.rsqrt(var + EPS) * g1_ref[...] + c1_ref[...], 0.0)
    x = jnp.dot(x, w2_ref[...], preferred_element_type=jnp.float32,
                precision=lax.Precision.HIGHEST) + b2_ref[...]
    mu = jnp.mean(x, 0, keepdims=True)
    d = x - mu
    var = jnp.mean(d * d, 0, keepdims=True)
    x = jnp.maximum(d * lax.rsqrt(var + EPS) * g2_ref[...] + c2_ref[...], 0.0)
    x = jnp.dot(x, w3_ref[...], preferred_element_type=jnp.float32,
                precision=lax.Precision.HIGHEST) + b3_ref[...]
    o_ref[...] = 1.0 / (1.0 + jnp.exp(-x))


_head = pl.pallas_call(
    _head_body,
    out_shape=jax.ShapeDtypeStruct((G, H), jnp.float32),
)


def kernel(feats, edge_index, node_graph_ids,
           gcn1_W, gcn1_b, gcn1_bn_g, gcn1_bn_b,
           gcn2_0_W, gcn2_0_b, gcn2_0_bn_g, gcn2_0_bn_b,
           gcn2_1_W, gcn2_1_b, gcn2_1_bn_g, gcn2_1_bn_b,
           aw_W, aw_b, fc1_W, fc1_b, bn1_g, bn1_b,
           lin0_W, lin0_b, bnl0_g, bnl0_b, fc2_W, fc2_b):
    src = edge_index[0]
    dst = edge_index[1]
    feats64 = jnp.pad(feats, ((0, 0), (0, INP - IN)))
    W1p = jnp.pad(gcn1_W, ((0, INP - IN), (0, 0)))
    gids2 = node_graph_ids.reshape(N, 1)

    npad = EROWS * 128 - E
    pidx = jnp.arange(npad, dtype=jnp.int32)
    src2 = jnp.concatenate([src, pidx % 128]).reshape(EROWS, 128)
    dst2 = jnp.concatenate([dst, N + pidx % 48]).reshape(EROWS, 128)
    eidx = jnp.stack([src2, dst2], axis=1).reshape(-1)

    degflat = _deg_kernel(src, dst)
    degr = degflat.reshape(2, 2, DEGP)
    po0 = degr[0, 0, :N].reshape(N, 1)
    pi0 = degr[0, 1, :N].reshape(N, 1)
    po1 = degr[1, 0, :N].reshape(N, 1)
    pi1 = degr[1, 1, :N].reshape(N, 1)

    t0a, t0b, ns, nd = _k0(feats64, po0, po1, pi0, pi1)

    u1 = _edge_scatter2(eidx, t0a, t0b)
    z1, st1 = _ka64(*u1, nd, W1p, gcn1_b.reshape(1, H))
    vs1 = _kv(z1, st1)
    t1 = _kb_mid(z1, st1, vs1, ns, gcn1_bn_g.reshape(1, H),
                 gcn1_bn_b.reshape(1, H))

    u2 = _edge_scatter4(eidx, *t1)
    z2, st2 = _ka128(*u2, nd, gcn2_0_W, gcn2_0_b.reshape(1, H))
    vs2 = _kv(z2, st2)
    t2 = _kb_mid(z2, st2, vs2, ns, gcn2_0_bn_g.reshape(1, H),
                 gcn2_0_bn_b.reshape(1, H))

    u3 = _edge_scatter4(eidx, *t2)
    z3, st3 = _ka128(*u3, nd, gcn2_1_W, gcn2_1_b.reshape(1, H))
    vs3 = _kv(z3, st3)
    aw, seg = _kb_fin(z3, st3, vs3, gcn2_1_bn_g.reshape(1, H),
                      gcn2_1_bn_b.reshape(1, H), aw_W.reshape(1, H),
                      aw_b.reshape(1, 1), gids2)

    w3p = jnp.pad(fc2_W, ((0, 0), (0, H - OUT)))
    b3p = jnp.pad(fc2_b, ((0, H - OUT))).reshape(1, H)
    headp = _head(seg, fc1_W, fc1_b.reshape(1, 256), bn1_g.reshape(1, 256),
                  bn1_b.reshape(1, 256), lin0_W, lin0_b.reshape(1, H),
                  bnl0_g.reshape(1, H), bnl0_b.reshape(1, H), w3p, b3p)
    x = headp[:, :OUT]
    return (x, aw)
